# speedup vs baseline: 4.3131x; 4.3131x over previous
import functools

import jax
import jax.numpy as jnp
from jax import lax
from jax.experimental import pallas as pl
from jax.experimental.pallas import tpu as pltpu
from jax.experimental.pallas import tpu_sc as plsc

NUM_NEURONS = 10000
PERIOD = 32
D_FEAT = 128
LANES = 16
NW = 32
CH = 8
ROWS = CH * PERIOD
NCHUNK = NUM_NEURONS // CH
STEPS = (NCHUNK + NW - 1) // NW
NJ = D_FEAT // LANES


def _sc_segmean(source, idx):
    mesh = plsc.VectorSubcoreMesh(core_axis_name="c", subcore_axis_name="s")

    @functools.partial(
        pl.kernel,
        mesh=mesh,
        out_type=jax.ShapeDtypeStruct((NUM_NEURONS, D_FEAT), jnp.float32),
        scratch_types=[
            pltpu.VMEM((ROWS,), jnp.int32),
            pltpu.VMEM((ROWS, D_FEAT), jnp.float32),
            pltpu.VMEM((CH, D_FEAT), jnp.float32),
            pltpu.SemaphoreType.DMA,
        ],
    )
    def k(src_hbm, idx_hbm, out_hbm, idx_v, rows_v, out_v, sem):
        wid = lax.axis_index("s") * 2 + lax.axis_index("c")

        @pl.loop(0, STEPS)
        def _step(step):
            chunk = step * NW + wid

            @pl.when(chunk < NCHUNK)
            def _():
                pltpu.sync_copy(idx_hbm.at[pl.ds(chunk * ROWS, ROWS)], idx_v)
                pltpu.async_copy(src_hbm.at[idx_v], rows_v, sem).wait()
                for o in range(CH):
                    def body(r, accs, o=o):
                        return tuple(
                            accs[j] + rows_v[o * PERIOD + r, pl.ds(j * LANES, LANES)]
                            for j in range(NJ)
                        )
                    accs = lax.fori_loop(
                        0, PERIOD, body,
                        tuple(jnp.zeros((LANES,), jnp.float32) for _ in range(NJ)))
                    for j in range(NJ):
                        out_v[o, pl.ds(j * LANES, LANES)] = accs[j] * (1.0 / PERIOD)
                pltpu.sync_copy(out_v, out_hbm.at[pl.ds(chunk * CH, CH)])

    return k(source, idx)


def kernel(source, gather_indices):
    return _sc_segmean(source, gather_indices.astype(jnp.int32))


# speedup vs baseline: 8.0375x; 1.8635x over previous
import functools

import jax
import jax.numpy as jnp
from jax import lax
from jax.experimental import pallas as pl
from jax.experimental.pallas import tpu as pltpu
from jax.experimental.pallas import tpu_sc as plsc

NUM_NEURONS = 10000
PERIOD = 32
D_FEAT = 128
LANES = 16
NW = 32
CH = 8
ROWS = CH * PERIOD
NCHUNK = NUM_NEURONS // CH
STEPS = (NCHUNK + NW - 1) // NW
NJ = D_FEAT // LANES


def _sc_segmean(source, idx):
    mesh = plsc.VectorSubcoreMesh(core_axis_name="c", subcore_axis_name="s")

    @functools.partial(
        pl.kernel,
        mesh=mesh,
        out_type=jax.ShapeDtypeStruct((NUM_NEURONS, D_FEAT), jnp.float32),
        scratch_types=[
            pltpu.VMEM((ROWS,), jnp.int32),
            pltpu.VMEM((ROWS,), jnp.int32),
            pltpu.VMEM((ROWS, D_FEAT), jnp.float32),
            pltpu.VMEM((ROWS, D_FEAT), jnp.float32),
            pltpu.VMEM((CH, D_FEAT), jnp.float32),
            pltpu.VMEM((CH, D_FEAT), jnp.float32),
            pltpu.SemaphoreType.DMA,
            pltpu.SemaphoreType.DMA,
            pltpu.SemaphoreType.DMA,
            pltpu.SemaphoreType.DMA,
            pltpu.SemaphoreType.DMA,
            pltpu.SemaphoreType.DMA,
        ],
    )
    def k(src_hbm, idx_hbm, out_hbm,
          idx_v0, idx_v1, rows_v0, rows_v1, out_v0, out_v1,
          isem0, isem1, gsem0, gsem1, osem0, osem1):
        idx_v = (idx_v0, idx_v1)
        rows_v = (rows_v0, rows_v1)
        out_v = (out_v0, out_v1)
        isem = (isem0, isem1)
        gsem = (gsem0, gsem1)
        osem = (osem0, osem1)
        wid = lax.axis_index("s") * 2 + lax.axis_index("c")

        for b in range(2):
            chunk = b * NW + wid
            pltpu.sync_copy(idx_hbm.at[pl.ds(chunk * ROWS, ROWS)], idx_v[b])
            pltpu.async_copy(src_hbm.at[idx_v[b]], rows_v[b], gsem[b])

        def accumulate(b):
            for o in range(CH):
                def body(r, accs, o=o, b=b):
                    return tuple(
                        accs[j] + rows_v[b][o * PERIOD + r, pl.ds(j * LANES, LANES)]
                        for j in range(NJ)
                    )
                accs = lax.fori_loop(
                    0, PERIOD, body,
                    tuple(jnp.zeros((LANES,), jnp.float32) for _ in range(NJ)))
                for j in range(NJ):
                    out_v[b][o, pl.ds(j * LANES, LANES)] = accs[j] * (1.0 / PERIOD)

        @pl.loop(0, STEPS // 2)
        def _t(t):
            for b in range(2):
                step = t * 2 + b
                chunk = step * NW + wid
                chunk2 = chunk + 2 * NW

                @pl.when(chunk < NCHUNK)
                def _():
                    pltpu.make_async_copy(
                        src_hbm.at[idx_v[b]], rows_v[b], gsem[b]).wait()

                    @pl.when(chunk2 < NCHUNK)
                    def _():
                        pltpu.async_copy(
                            idx_hbm.at[pl.ds(chunk2 * ROWS, ROWS)],
                            idx_v[b], isem[b])

                    @pl.when(t > 0)
                    def _():
                        pltpu.make_async_copy(
                            out_v[b], out_hbm.at[pl.ds(0, CH)], osem[b]).wait()

                    accumulate(b)

                    @pl.when(chunk2 < NCHUNK)
                    def _():
                        pltpu.make_async_copy(
                            idx_hbm.at[pl.ds(chunk2 * ROWS, ROWS)],
                            idx_v[b], isem[b]).wait()
                        pltpu.async_copy(src_hbm.at[idx_v[b]], rows_v[b], gsem[b])

                    pltpu.async_copy(
                        out_v[b], out_hbm.at[pl.ds(chunk * CH, CH)], osem[b])

        for b in range(2):
            pltpu.make_async_copy(out_v[b], out_hbm.at[pl.ds(0, CH)], osem[b]).wait()

    return k(source, idx)


def kernel(source, gather_indices):
    return _sc_segmean(source, gather_indices.astype(jnp.int32))
